# baseline (speedup 1.0000x reference)
import functools
import math

import jax
import jax.numpy as jnp
from jax.experimental import pallas as pl
from jax.experimental.pallas import tpu as pltpu


def _router_block(x_ref, w_ref, idx_ref, loc_ref, gate_ref, carry_ref,
                  tri_ref, *, bt, num_experts):
    b = pl.program_id(0)

    @pl.when(b == 0)
    def _():
        carry_ref[...] = jnp.zeros_like(carry_ref)
        ti = jax.lax.broadcasted_iota(jnp.int32, (bt, bt), 0)
        tj = jax.lax.broadcasted_iota(jnp.int32, (bt, bt), 1)
        tri_ref[...] = (tj < ti).astype(jnp.float32)

    x = x_ref[...]
    w = w_ref[...]
    logits = jax.lax.dot_general(
        x, w, (((1,), (1,)), ((), ())), preferred_element_type=jnp.float32)

    m = jnp.max(logits, axis=1, keepdims=True)
    eiota = jax.lax.broadcasted_iota(jnp.int32, logits.shape, 1)
    idx = jnp.min(jnp.where(logits == m, eiota, num_experts), axis=1)
    gate = 1.0 / jnp.sum(jnp.exp(logits - m), axis=1)

    maskf = (eiota == idx[:, None]).astype(jnp.float32)
    prev = jax.lax.dot(tri_ref[...], maskf, preferred_element_type=jnp.float32)

    carry = carry_ref[...]
    loc = jnp.sum((prev + carry) * maskf, axis=1)
    carry_ref[...] = carry + jnp.sum(maskf, axis=0, keepdims=True)

    idx_ref[...] = idx.reshape(1, 1, bt)
    loc_ref[...] = loc.astype(jnp.int32).reshape(1, 1, bt)
    gate_ref[...] = gate.reshape(1, 1, bt)


def kernel(input, W):
    tokens, model_dim = input.shape
    num_experts = W.shape[0]
    bt = min(256, tokens)
    nblk = tokens // bt
    capacity = int(math.ceil(tokens / num_experts))

    body = functools.partial(_router_block, bt=bt, num_experts=num_experts)
    idx, loc, gate = pl.pallas_call(
        body,
        grid=(nblk,),
        in_specs=[
            pl.BlockSpec((bt, model_dim), lambda i: (i, 0)),
            pl.BlockSpec((num_experts, model_dim), lambda i: (0, 0)),
        ],
        out_specs=[
            pl.BlockSpec((1, 1, bt), lambda i: (i, 0, 0)),
            pl.BlockSpec((1, 1, bt), lambda i: (i, 0, 0)),
            pl.BlockSpec((1, 1, bt), lambda i: (i, 0, 0)),
        ],
        out_shape=[
            jax.ShapeDtypeStruct((nblk, 1, bt), jnp.int32),
            jax.ShapeDtypeStruct((nblk, 1, bt), jnp.int32),
            jax.ShapeDtypeStruct((nblk, 1, bt), jnp.float32),
        ],
        scratch_shapes=[pltpu.VMEM((1, num_experts), jnp.float32),
                        pltpu.VMEM((bt, bt), jnp.float32)],
    )(input, W)

    return (idx.reshape(tokens), jnp.int32(capacity),
            loc.reshape(tokens), gate.reshape(tokens),
            jnp.int32(num_experts))


# speedup vs baseline: 1.0444x; 1.0444x over previous
import functools
import math

import jax
import jax.numpy as jnp
from jax import lax
from jax.experimental import pallas as pl
from jax.experimental.pallas import tpu as pltpu
from jax.experimental.pallas import tpu_sc as plsc

_NC = 2
_NS = 16
_NW = _NC * _NS
_L = 16


def _gate_block(x_ref, w_ref, idx_ref, gate_ref, *, bt, num_experts):
    x = x_ref[...]
    w = w_ref[...]
    logits = jax.lax.dot_general(
        x, w, (((1,), (1,)), ((), ())), preferred_element_type=jnp.float32)

    m = jnp.max(logits, axis=1, keepdims=True)
    eiota = jax.lax.broadcasted_iota(jnp.int32, logits.shape, 1)
    idx = jnp.min(jnp.where(logits == m, eiota, num_experts), axis=1)
    gate = 1.0 / jnp.sum(jnp.exp(logits - m), axis=1)

    idx_ref[...] = idx.reshape(1, 1, bt)
    gate_ref[...] = gate.reshape(1, 1, bt)


def _tc_gate(input, W, bt):
    tokens, model_dim = input.shape
    num_experts = W.shape[0]
    nblk = tokens // bt
    body = functools.partial(_gate_block, bt=bt, num_experts=num_experts)
    return pl.pallas_call(
        body,
        grid=(nblk,),
        in_specs=[
            pl.BlockSpec((bt, model_dim), lambda i: (i, 0)),
            pl.BlockSpec((num_experts, model_dim), lambda i: (0, 0)),
        ],
        out_specs=[
            pl.BlockSpec((1, 1, bt), lambda i: (i, 0, 0)),
            pl.BlockSpec((1, 1, bt), lambda i: (i, 0, 0)),
        ],
        out_shape=[
            jax.ShapeDtypeStruct((nblk, 1, bt), jnp.int32),
            jax.ShapeDtypeStruct((nblk, 1, bt), jnp.float32),
        ],
    )(input, W)


def _make_sc_kernels(tokens, num_experts):
    tpw = tokens // _NW
    ngrp = tpw // _L
    mesh = plsc.VectorSubcoreMesh(core_axis_name="c", subcore_axis_name="s")

    @functools.partial(
        pl.kernel, mesh=mesh,
        compiler_params=pltpu.CompilerParams(needs_layout_passes=False),
        out_type=[jax.ShapeDtypeStruct((tokens,), jnp.int32),
                  jax.ShapeDtypeStruct((_NW, num_experts), jnp.int32)],
        scratch_types=[pltpu.VMEM((tpw,), jnp.int32),
                       pltpu.VMEM((tpw,), jnp.int32),
                       pltpu.VMEM((num_experts,), jnp.int32)],
    )
    def sc_pos_local(idx_hbm, pos_hbm, hist_hbm, idx_v, pos_v, hist_v):
        w = lax.axis_index("s") * _NC + lax.axis_index("c")
        base = w * tpw
        pltpu.sync_copy(idx_hbm.at[pl.ds(base, tpw)], idx_v)
        zeros16 = jnp.zeros((_L,), jnp.int32)
        for j in range(num_experts // _L):
            hist_v[pl.ds(j * _L, _L)] = zeros16
        iota16 = lax.iota(jnp.int32, _L)

        def body(g, carry):
            idx16 = idx_v[pl.ds(g * _L, _L)]
            rank = jnp.zeros((_L,), jnp.int32)
            fwd = jnp.zeros((_L,), jnp.int32)
            for s in range(1, _L):
                jb = jnp.maximum(iota16 - s, 0)
                xb = idx16.at[jb].get(mode="promise_in_bounds")
                rank = rank + (jnp.where(iota16 >= s, 1, 0) *
                               jnp.where(xb == idx16, 1, 0))
                jf = jnp.minimum(iota16 + s, _L - 1)
                xf = idx16.at[jf].get(mode="promise_in_bounds")
                fwd = fwd + (jnp.where(iota16 < _L - s, 1, 0) *
                             jnp.where(xf == idx16, 1, 0))
            old = plsc.load_gather(hist_v, [idx16])
            pos_v[pl.ds(g * _L, _L)] = old + rank
            plsc.store_scatter(hist_v, [idx16], old + rank + 1,
                               mask=(fwd == 0))
            return carry

        lax.fori_loop(0, ngrp, body, 0)
        pltpu.sync_copy(pos_v, pos_hbm.at[pl.ds(base, tpw)])
        pltpu.sync_copy(hist_v, hist_hbm.at[w])

    @functools.partial(
        pl.kernel, mesh=mesh,
        compiler_params=pltpu.CompilerParams(needs_layout_passes=False),
        out_type=jax.ShapeDtypeStruct((tokens,), jnp.int32),
        scratch_types=[pltpu.VMEM((_NW, num_experts), jnp.int32),
                       pltpu.VMEM((tpw,), jnp.int32),
                       pltpu.VMEM((tpw,), jnp.int32),
                       pltpu.VMEM((num_experts,), jnp.int32),
                       pltpu.VMEM((tpw,), jnp.int32)],
    )
    def sc_pos_global(idx_hbm, pos_hbm, hist_hbm, loc_hbm,
                      hist_v, idx_v, pos_v, offs_v, loc_v):
        w = lax.axis_index("s") * _NC + lax.axis_index("c")
        base = w * tpw
        pltpu.sync_copy(hist_hbm, hist_v)
        pltpu.sync_copy(idx_hbm.at[pl.ds(base, tpw)], idx_v)
        pltpu.sync_copy(pos_hbm.at[pl.ds(base, tpw)], pos_v)
        for j in range(num_experts // _L):
            acc = jnp.zeros((_L,), jnp.int32)
            for wp in range(_NW):
                row = hist_v[wp, pl.ds(j * _L, _L)]
                acc = acc + row * jnp.where(wp < w, 1, 0)
            offs_v[pl.ds(j * _L, _L)] = acc

        def body(g, carry):
            idx16 = idx_v[pl.ds(g * _L, _L)]
            loc_v[pl.ds(g * _L, _L)] = (
                pos_v[pl.ds(g * _L, _L)] + plsc.load_gather(offs_v, [idx16]))
            return carry

        lax.fori_loop(0, ngrp, body, 0)
        pltpu.sync_copy(loc_v, loc_hbm.at[pl.ds(base, tpw)])

    return sc_pos_local, sc_pos_global


def kernel(input, W):
    tokens, model_dim = input.shape
    num_experts = W.shape[0]
    bt = min(512, tokens)
    capacity = int(math.ceil(tokens / num_experts))

    idx3, gate3 = _tc_gate(input, W, bt)
    idx = idx3.reshape(tokens)

    sc_pos_local, sc_pos_global = _make_sc_kernels(tokens, num_experts)
    pos_local, hist = sc_pos_local(idx)
    loc = sc_pos_global(idx, pos_local, hist)

    return (idx, jnp.int32(capacity), loc, gate3.reshape(tokens),
            jnp.int32(num_experts))


# speedup vs baseline: 1.0661x; 1.0208x over previous
import functools
import math

import jax
import jax.numpy as jnp
from jax import lax
from jax.experimental import pallas as pl
from jax.experimental.pallas import tpu as pltpu
from jax.experimental.pallas import tpu_sc as plsc

_NC = 2
_NS = 16
_NW = _NC * _NS
_L = 16


def _gate_block(x_ref, w_ref, idx_ref, gate_ref, *, bt, num_experts):
    x = x_ref[...]
    w = w_ref[...]
    logits = jax.lax.dot_general(
        x, w, (((1,), (1,)), ((), ())), preferred_element_type=jnp.float32)

    m = jnp.max(logits, axis=1, keepdims=True)
    eiota = jax.lax.broadcasted_iota(jnp.int32, logits.shape, 1)
    idx = jnp.min(jnp.where(logits == m, eiota, num_experts), axis=1)
    gate = 1.0 / jnp.sum(jnp.exp(logits - m), axis=1)

    idx_ref[...] = idx.reshape(1, 1, bt)
    gate_ref[...] = gate.reshape(1, 1, bt)


def _tc_gate(input, W, bt):
    tokens, model_dim = input.shape
    num_experts = W.shape[0]
    nblk = tokens // bt
    body = functools.partial(_gate_block, bt=bt, num_experts=num_experts)
    return pl.pallas_call(
        body,
        grid=(nblk,),
        in_specs=[
            pl.BlockSpec((bt, model_dim), lambda i: (i, 0)),
            pl.BlockSpec((num_experts, model_dim), lambda i: (0, 0)),
        ],
        out_specs=[
            pl.BlockSpec((1, 1, bt), lambda i: (i, 0, 0)),
            pl.BlockSpec((1, 1, bt), lambda i: (i, 0, 0)),
        ],
        out_shape=[
            jax.ShapeDtypeStruct((nblk, 1, bt), jnp.int32),
            jax.ShapeDtypeStruct((nblk, 1, bt), jnp.float32),
        ],
    )(input, W)


def _make_sc_1core(tokens, num_experts):
    tpw = tokens // _NS
    ngrp = tpw // _L
    mesh = plsc.VectorSubcoreMesh(core_axis_name="c", subcore_axis_name="s")

    @functools.partial(
        pl.kernel, mesh=mesh,
        compiler_params=pltpu.CompilerParams(needs_layout_passes=False),
        out_type=jax.ShapeDtypeStruct((tokens,), jnp.int32),
        scratch_types=[pltpu.VMEM((tpw,), jnp.int32),
                       pltpu.VMEM((tpw,), jnp.int32),
                       pltpu.VMEM((num_experts,), jnp.int32),
                       pltpu.VMEM((_NS * num_experts,), jnp.int32),
                       pltpu.VMEM((num_experts,), jnp.int32),
                       pltpu.VMEM_SHARED((_NS * num_experts,), jnp.int32)],
    )
    def sc_loc(idx_hbm, loc_hbm, idx_v, pos_v, hist_v, rows_v, offs_v,
               shared):
        c = lax.axis_index("c")
        s = lax.axis_index("s")

        @pl.when(c == 0)
        def _():
            base = s * tpw
            pltpu.sync_copy(idx_hbm.at[pl.ds(base, tpw)], idx_v)
            zeros16 = jnp.zeros((_L,), jnp.int32)
            for j in range(num_experts // _L):
                hist_v[pl.ds(j * _L, _L)] = zeros16
            iota16 = lax.iota(jnp.int32, _L)

            def body(g, carry):
                idx16 = idx_v[pl.ds(g * _L, _L)]
                rank = jnp.zeros((_L,), jnp.int32)
                fwd = jnp.zeros((_L,), jnp.int32)
                for sh in range(1, _L):
                    jb = jnp.maximum(iota16 - sh, 0)
                    xb = idx16.at[jb].get(mode="promise_in_bounds")
                    rank = rank + (jnp.where(iota16 >= sh, 1, 0) *
                                   jnp.where(xb == idx16, 1, 0))
                    jf = jnp.minimum(iota16 + sh, _L - 1)
                    xf = idx16.at[jf].get(mode="promise_in_bounds")
                    fwd = fwd + (jnp.where(iota16 < _L - sh, 1, 0) *
                                 jnp.where(xf == idx16, 1, 0))
                old = plsc.load_gather(hist_v, [idx16])
                pos_v[pl.ds(g * _L, _L)] = old + rank
                plsc.store_scatter(hist_v, [idx16], old + rank + 1,
                                   mask=(fwd == 0))
                return carry

            lax.fori_loop(0, ngrp, body, 0)
            pltpu.sync_copy(hist_v, shared.at[pl.ds(s * num_experts,
                                                    num_experts)])
            plsc.subcore_barrier()
            pltpu.sync_copy(shared, rows_v)
            for j in range(num_experts // _L):
                acc = jnp.zeros((_L,), jnp.int32)
                for sp in range(_NS):
                    acc = acc + (rows_v[pl.ds(sp * num_experts + j * _L, _L)] *
                                 jnp.where(sp < s, 1, 0))
                offs_v[pl.ds(j * _L, _L)] = acc

            def obody(g, carry):
                idx16 = idx_v[pl.ds(g * _L, _L)]
                pos_v[pl.ds(g * _L, _L)] = (
                    pos_v[pl.ds(g * _L, _L)] +
                    plsc.load_gather(offs_v, [idx16]))
                return carry

            lax.fori_loop(0, ngrp, obody, 0)
            pltpu.sync_copy(pos_v, loc_hbm.at[pl.ds(base, tpw)])

    return sc_loc


def _make_sc_kernels(tokens, num_experts):
    tpw = tokens // _NW
    ngrp = tpw // _L
    mesh = plsc.VectorSubcoreMesh(core_axis_name="c", subcore_axis_name="s")

    @functools.partial(
        pl.kernel, mesh=mesh,
        compiler_params=pltpu.CompilerParams(needs_layout_passes=False),
        out_type=[jax.ShapeDtypeStruct((tokens,), jnp.int32),
                  jax.ShapeDtypeStruct((_NW, num_experts), jnp.int32)],
        scratch_types=[pltpu.VMEM((tpw,), jnp.int32),
                       pltpu.VMEM((tpw,), jnp.int32),
                       pltpu.VMEM((num_experts,), jnp.int32)],
    )
    def sc_pos_local(idx_hbm, pos_hbm, hist_hbm, idx_v, pos_v, hist_v):
        w = lax.axis_index("s") * _NC + lax.axis_index("c")
        base = w * tpw
        pltpu.sync_copy(idx_hbm.at[pl.ds(base, tpw)], idx_v)
        zeros16 = jnp.zeros((_L,), jnp.int32)
        for j in range(num_experts // _L):
            hist_v[pl.ds(j * _L, _L)] = zeros16
        iota16 = lax.iota(jnp.int32, _L)

        def body(g, carry):
            idx16 = idx_v[pl.ds(g * _L, _L)]
            rank = jnp.zeros((_L,), jnp.int32)
            fwd = jnp.zeros((_L,), jnp.int32)
            for s in range(1, _L):
                jb = jnp.maximum(iota16 - s, 0)
                xb = idx16.at[jb].get(mode="promise_in_bounds")
                rank = rank + (jnp.where(iota16 >= s, 1, 0) *
                               jnp.where(xb == idx16, 1, 0))
                jf = jnp.minimum(iota16 + s, _L - 1)
                xf = idx16.at[jf].get(mode="promise_in_bounds")
                fwd = fwd + (jnp.where(iota16 < _L - s, 1, 0) *
                             jnp.where(xf == idx16, 1, 0))
            old = plsc.load_gather(hist_v, [idx16])
            pos_v[pl.ds(g * _L, _L)] = old + rank
            plsc.store_scatter(hist_v, [idx16], old + rank + 1,
                               mask=(fwd == 0))
            return carry

        lax.fori_loop(0, ngrp, body, 0)
        pltpu.sync_copy(pos_v, pos_hbm.at[pl.ds(base, tpw)])
        pltpu.sync_copy(hist_v, hist_hbm.at[w])

    @functools.partial(
        pl.kernel, mesh=mesh,
        compiler_params=pltpu.CompilerParams(needs_layout_passes=False),
        out_type=jax.ShapeDtypeStruct((tokens,), jnp.int32),
        scratch_types=[pltpu.VMEM((_NW, num_experts), jnp.int32),
                       pltpu.VMEM((tpw,), jnp.int32),
                       pltpu.VMEM((tpw,), jnp.int32),
                       pltpu.VMEM((num_experts,), jnp.int32),
                       pltpu.VMEM((tpw,), jnp.int32)],
    )
    def sc_pos_global(idx_hbm, pos_hbm, hist_hbm, loc_hbm,
                      hist_v, idx_v, pos_v, offs_v, loc_v):
        w = lax.axis_index("s") * _NC + lax.axis_index("c")
        base = w * tpw
        pltpu.sync_copy(hist_hbm, hist_v)
        pltpu.sync_copy(idx_hbm.at[pl.ds(base, tpw)], idx_v)
        pltpu.sync_copy(pos_hbm.at[pl.ds(base, tpw)], pos_v)
        for j in range(num_experts // _L):
            acc = jnp.zeros((_L,), jnp.int32)
            for wp in range(_NW):
                row = hist_v[wp, pl.ds(j * _L, _L)]
                acc = acc + row * jnp.where(wp < w, 1, 0)
            offs_v[pl.ds(j * _L, _L)] = acc

        def body(g, carry):
            idx16 = idx_v[pl.ds(g * _L, _L)]
            loc_v[pl.ds(g * _L, _L)] = (
                pos_v[pl.ds(g * _L, _L)] + plsc.load_gather(offs_v, [idx16]))
            return carry

        lax.fori_loop(0, ngrp, body, 0)
        pltpu.sync_copy(loc_v, loc_hbm.at[pl.ds(base, tpw)])

    return sc_pos_local, sc_pos_global


def kernel(input, W):
    tokens, model_dim = input.shape
    num_experts = W.shape[0]
    bt = min(512, tokens)
    capacity = int(math.ceil(tokens / num_experts))

    idx3, gate3 = _tc_gate(input, W, bt)
    idx = idx3.reshape(tokens)

    sc_loc = _make_sc_1core(tokens, num_experts)
    loc = sc_loc(idx)

    return (idx, jnp.int32(capacity), loc, gate3.reshape(tokens),
            jnp.int32(num_experts))


# speedup vs baseline: 1.0696x; 1.0033x over previous
import functools
import math

import jax
import jax.numpy as jnp
from jax import lax
from jax.experimental import pallas as pl
from jax.experimental.pallas import tpu as pltpu
from jax.experimental.pallas import tpu_sc as plsc

_NC = 2
_NS = 16
_NW = _NC * _NS
_L = 16


def _gate_block(x_ref, w_ref, idx_ref, gate_ref, *, bt, num_experts):
    x = x_ref[...]
    w = w_ref[...]
    logits = jax.lax.dot_general(
        x, w, (((1,), (1,)), ((), ())), preferred_element_type=jnp.float32)

    m = jnp.max(logits, axis=1, keepdims=True)
    eiota = jax.lax.broadcasted_iota(jnp.int32, logits.shape, 1)
    idx = jnp.min(jnp.where(logits == m, eiota, num_experts), axis=1)
    gate = 1.0 / jnp.sum(jnp.exp(logits - m), axis=1)

    idx_ref[...] = idx.reshape(1, 1, bt)
    gate_ref[...] = gate.reshape(1, 1, bt)


def _tc_gate(input, W, bt):
    tokens, model_dim = input.shape
    num_experts = W.shape[0]
    nblk = tokens // bt
    body = functools.partial(_gate_block, bt=bt, num_experts=num_experts)
    return pl.pallas_call(
        body,
        grid=(nblk,),
        in_specs=[
            pl.BlockSpec((bt, model_dim), lambda i: (i, 0)),
            pl.BlockSpec((num_experts, model_dim), lambda i: (0, 0)),
        ],
        out_specs=[
            pl.BlockSpec((1, 1, bt), lambda i: (i, 0, 0)),
            pl.BlockSpec((1, 1, bt), lambda i: (i, 0, 0)),
        ],
        out_shape=[
            jax.ShapeDtypeStruct((nblk, 1, bt), jnp.int32),
            jax.ShapeDtypeStruct((nblk, 1, bt), jnp.float32),
        ],
    )(input, W)


def _make_sc_1core(tokens, num_experts):
    tpw = tokens // _NS
    ngrp = tpw // _L
    mesh = plsc.VectorSubcoreMesh(core_axis_name="c", subcore_axis_name="s")

    @functools.partial(
        pl.kernel, mesh=mesh,
        compiler_params=pltpu.CompilerParams(needs_layout_passes=False),
        out_type=jax.ShapeDtypeStruct((tokens,), jnp.int32),
        scratch_types=[pltpu.VMEM((tpw,), jnp.int32),
                       pltpu.VMEM((tpw,), jnp.int32),
                       pltpu.VMEM((num_experts,), jnp.int32),
                       pltpu.VMEM((_NS * num_experts,), jnp.int32),
                       pltpu.VMEM((num_experts,), jnp.int32),
                       pltpu.VMEM_SHARED((_NS * num_experts,), jnp.int32)],
    )
    def sc_loc(idx_hbm, loc_hbm, idx_v, pos_v, hist_v, rows_v, offs_v,
               shared):
        c = lax.axis_index("c")
        s = lax.axis_index("s")

        @pl.when(c == 0)
        def _():
            base = s * tpw
            pltpu.sync_copy(idx_hbm.at[pl.ds(base, tpw)], idx_v)
            zeros16 = jnp.zeros((_L,), jnp.int32)
            for j in range(num_experts // _L):
                hist_v[pl.ds(j * _L, _L)] = zeros16
            iota16 = lax.iota(jnp.int32, _L)

            def body(g, carry):
                idx16 = idx_v[pl.ds(g * _L, _L)]
                rank = jnp.zeros((_L,), jnp.int32)
                fwd = jnp.zeros((_L,), jnp.int32)
                for sh in range(1, _L):
                    jb = jnp.maximum(iota16 - sh, 0)
                    xb = idx16.at[jb].get(mode="promise_in_bounds")
                    rank = rank + (jnp.where(iota16 >= sh, 1, 0) *
                                   jnp.where(xb == idx16, 1, 0))
                    jf = jnp.minimum(iota16 + sh, _L - 1)
                    xf = idx16.at[jf].get(mode="promise_in_bounds")
                    fwd = fwd + (jnp.where(iota16 < _L - sh, 1, 0) *
                                 jnp.where(xf == idx16, 1, 0))
                old = plsc.load_gather(hist_v, [idx16])
                pos_v[pl.ds(g * _L, _L)] = old + rank
                plsc.store_scatter(hist_v, [idx16], old + rank + 1,
                                   mask=(fwd == 0))
                return carry

            lax.fori_loop(0, ngrp, body, 0)
            pltpu.sync_copy(hist_v, shared.at[pl.ds(s * num_experts,
                                                    num_experts)])
            plsc.subcore_barrier()
            pltpu.sync_copy(shared, rows_v)
            for j in range(num_experts // _L):
                acc = jnp.zeros((_L,), jnp.int32)
                for sp in range(_NS):
                    acc = acc + (rows_v[pl.ds(sp * num_experts + j * _L, _L)] *
                                 jnp.where(sp < s, 1, 0))
                offs_v[pl.ds(j * _L, _L)] = acc

            def obody(g, carry):
                idx16 = idx_v[pl.ds(g * _L, _L)]
                pos_v[pl.ds(g * _L, _L)] = (
                    pos_v[pl.ds(g * _L, _L)] +
                    plsc.load_gather(offs_v, [idx16]))
                return carry

            lax.fori_loop(0, ngrp, obody, 0)
            pltpu.sync_copy(pos_v, loc_hbm.at[pl.ds(base, tpw)])

    return sc_loc


def kernel(input, W):
    tokens, model_dim = input.shape
    num_experts = W.shape[0]
    bt = min(512, tokens)
    capacity = int(math.ceil(tokens / num_experts))

    idx3, gate3 = _tc_gate(input, W, bt)
    idx = idx3.reshape(tokens)

    sc_loc = _make_sc_1core(tokens, num_experts)
    loc = sc_loc(idx)

    return (idx, jnp.int32(capacity), loc, gate3.reshape(tokens),
            jnp.int32(num_experts))
